# Initial kernel scaffold; baseline (speedup 1.0000x reference)
import jax
import jax.numpy as jnp
from jax.experimental import pallas as pl


def kernel(x, edge_index, W1, b1, W2, b2):
    raise NotImplementedError("write your pallas kernel here")


# speedup vs baseline: 5.4591x; 5.4591x over previous
import functools

import jax
import jax.numpy as jnp
from jax import lax
from jax.experimental import pallas as pl
from jax.experimental.pallas import tpu as pltpu
from jax.experimental.pallas import tpu_sc as plsc

_N = 10000
_NP = 10240
_E = 320000
_D = 128
_NC = 2
_NS = 16
_CH = 80


def _make_sc_agg():
    mesh = plsc.VectorSubcoreMesh(core_axis_name="c", subcore_axis_name="s")
    n_workers = _NC * _NS
    epw = _E // n_workers
    n_chunks = epw // _CH
    rows_per_tile = _NP // _NS

    @functools.partial(
        pl.kernel,
        mesh=mesh,
        out_type=jax.ShapeDtypeStruct((_NC, _NP, _D), jnp.float32),
        scratch_types=[
            pltpu.VMEM((_CH,), jnp.int32),
            pltpu.VMEM((_CH,), jnp.int32),
            pltpu.VMEM((_CH, _D), jnp.float32),
            pltpu.VMEM_SHARED((_NP, _D), jnp.float32),
            pltpu.SemaphoreType.DMA,
        ],
    )
    def sc_agg(x_hbm, src_hbm, dst_hbm, zeros_hbm, out_hbm,
               src_v, dst_v, rows_v, agg_sh, sem):
        cid = lax.axis_index("c")
        sid = lax.axis_index("s")
        wid = sid * _NC + cid
        r0 = sid * rows_per_tile
        pltpu.sync_copy(zeros_hbm.at[pl.ds(r0, rows_per_tile)],
                        agg_sh.at[pl.ds(r0, rows_per_tile)])
        plsc.subcore_barrier()

        base = wid * epw

        def body(c, carry):
            off = base + c * _CH
            pltpu.sync_copy(src_hbm.at[pl.ds(off, _CH)], src_v)
            pltpu.sync_copy(dst_hbm.at[pl.ds(off, _CH)], dst_v)
            pltpu.async_copy(x_hbm.at[src_v], rows_v, sem).wait()
            pltpu.sync_copy(rows_v, agg_sh.at[dst_v], add=True)
            return carry

        lax.fori_loop(0, n_chunks, body, 0)
        plsc.subcore_barrier()
        pltpu.sync_copy(agg_sh.at[pl.ds(r0, rows_per_tile)],
                        out_hbm.at[cid, pl.ds(r0, rows_per_tile)])

    return sc_agg


_sc_agg = _make_sc_agg()

_BLK = 1000


def _mlp_body(x_ref, a_ref, w1_ref, b1_ref, w2_ref, b2_ref, o_ref):
    h = x_ref[...] + a_ref[0] + a_ref[1]
    h = jnp.dot(h, w1_ref[...], preferred_element_type=jnp.float32) + b1_ref[...]
    h = jnp.maximum(h, 0.01 * h)
    h = jnp.dot(h, w2_ref[...], preferred_element_type=jnp.float32) + b2_ref[...]
    o_ref[...] = jnp.maximum(h, 0.01 * h)


def _tc_mlp(x, agg2, W1, b1, W2, b2):
    return pl.pallas_call(
        _mlp_body,
        grid=(_N // _BLK,),
        in_specs=[
            pl.BlockSpec((_BLK, _D), lambda i: (i, 0)),
            pl.BlockSpec((_NC, _BLK, _D), lambda i: (0, i, 0)),
            pl.BlockSpec((_D, _D), lambda i: (0, 0)),
            pl.BlockSpec((1, _D), lambda i: (0, 0)),
            pl.BlockSpec((_D, _D), lambda i: (0, 0)),
            pl.BlockSpec((1, _D), lambda i: (0, 0)),
        ],
        out_specs=pl.BlockSpec((_BLK, _D), lambda i: (i, 0)),
        out_shape=jax.ShapeDtypeStruct((_N, _D), jnp.float32),
    )(x, agg2, W1, b1.reshape(1, _D), W2, b2.reshape(1, _D))


def kernel(x, edge_index, W1, b1, W2, b2):
    src = edge_index[0]
    dst = edge_index[1]
    zeros = jnp.zeros((_NP, _D), jnp.float32)
    agg2 = _sc_agg(x, src, dst, zeros)
    return _tc_mlp(x, agg2, W1, b1, W2, b2)


# speedup vs baseline: 11.7610x; 2.1544x over previous
import functools

import jax
import jax.numpy as jnp
from jax import lax
from jax.experimental import pallas as pl
from jax.experimental.pallas import tpu as pltpu
from jax.experimental.pallas import tpu_sc as plsc

_N = 10000
_NP = 10240
_E = 320000
_D = 128
_NC = 2
_NS = 16
_CH = 80


def _make_sc_agg():
    mesh = plsc.VectorSubcoreMesh(core_axis_name="c", subcore_axis_name="s")
    n_workers = _NC * _NS
    epw = _E // n_workers
    n_chunks = epw // _CH
    rows_per_tile = _NP // _NS

    @functools.partial(
        pl.kernel,
        mesh=mesh,
        out_type=jax.ShapeDtypeStruct((_NC, _NP, _D), jnp.float32),
        scratch_types=[
            pltpu.VMEM((epw,), jnp.int32),
            pltpu.VMEM((epw,), jnp.int32),
            pltpu.VMEM((_CH,), jnp.int32),
            pltpu.VMEM((_CH,), jnp.int32),
            pltpu.VMEM((_CH, _D), jnp.float32),
            pltpu.VMEM((_CH, _D), jnp.float32),
            pltpu.VMEM_SHARED((_NP, _D), jnp.float32),
            pltpu.SemaphoreType.DMA,
            pltpu.SemaphoreType.DMA,
        ],
    )
    def sc_agg(x_hbm, src_hbm, dst_hbm, zeros_hbm, out_hbm,
               src_v, dst_v, dstc0, dstc1, rows0, rows1, agg_sh, sem0, sem1):
        cid = lax.axis_index("c")
        sid = lax.axis_index("s")
        wid = sid * _NC + cid
        r0 = sid * rows_per_tile
        pltpu.sync_copy(zeros_hbm.at[pl.ds(r0, rows_per_tile)],
                        agg_sh.at[pl.ds(r0, rows_per_tile)])
        base = wid * epw
        pltpu.sync_copy(src_hbm.at[pl.ds(base, epw)], src_v)
        pltpu.sync_copy(dst_hbm.at[pl.ds(base, epw)], dst_v)
        plsc.subcore_barrier()

        def sl(c):
            return pl.ds(pl.multiple_of(c * _CH, _CH), _CH)

        def copy_dst(c, dstc):
            off = pl.multiple_of(c * _CH, _CH)
            for j in range(_CH // 16):
                dstc[pl.ds(16 * j, 16)] = dst_v[pl.ds(off + 16 * j, 16)]

        pltpu.async_copy(x_hbm.at[src_v.at[sl(0)]], rows0, sem0)
        copy_dst(0, dstc0)

        def wait(rows, sem):
            pltpu.make_async_copy(x_hbm.at[src_v.at[sl(0)]], rows, sem).wait()

        def body(i, carry):
            c = 2 * i + 1
            pltpu.async_copy(x_hbm.at[src_v.at[sl(c)]], rows1, sem1)
            copy_dst(c, dstc1)
            wait(rows0, sem0)
            pltpu.sync_copy(rows0, agg_sh.at[dstc0], add=True)
            pltpu.async_copy(x_hbm.at[src_v.at[sl(c + 1)]], rows0, sem0)
            copy_dst(c + 1, dstc0)
            wait(rows1, sem1)
            pltpu.sync_copy(rows1, agg_sh.at[dstc1], add=True)
            return carry

        lax.fori_loop(0, (n_chunks - 1) // 2, body, 0)
        wait(rows0, sem0)
        pltpu.sync_copy(rows0, agg_sh.at[dstc0], add=True)
        plsc.subcore_barrier()
        pltpu.sync_copy(agg_sh.at[pl.ds(r0, rows_per_tile)],
                        out_hbm.at[cid, pl.ds(r0, rows_per_tile)])

    return sc_agg


_sc_agg = _make_sc_agg()

_BLK = 1000


def _mlp_body(x_ref, a_ref, w1_ref, b1_ref, w2_ref, b2_ref, o_ref):
    h = x_ref[...] + a_ref[0] + a_ref[1]
    h = jnp.dot(h, w1_ref[...], preferred_element_type=jnp.float32) + b1_ref[...]
    h = jnp.maximum(h, 0.01 * h)
    h = jnp.dot(h, w2_ref[...], preferred_element_type=jnp.float32) + b2_ref[...]
    o_ref[...] = jnp.maximum(h, 0.01 * h)


def _tc_mlp(x, agg2, W1, b1, W2, b2):
    return pl.pallas_call(
        _mlp_body,
        grid=(_N // _BLK,),
        in_specs=[
            pl.BlockSpec((_BLK, _D), lambda i: (i, 0)),
            pl.BlockSpec((_NC, _BLK, _D), lambda i: (0, i, 0)),
            pl.BlockSpec((_D, _D), lambda i: (0, 0)),
            pl.BlockSpec((1, _D), lambda i: (0, 0)),
            pl.BlockSpec((_D, _D), lambda i: (0, 0)),
            pl.BlockSpec((1, _D), lambda i: (0, 0)),
        ],
        out_specs=pl.BlockSpec((_BLK, _D), lambda i: (i, 0)),
        out_shape=jax.ShapeDtypeStruct((_N, _D), jnp.float32),
    )(x, agg2, W1, b1.reshape(1, _D), W2, b2.reshape(1, _D))


def kernel(x, edge_index, W1, b1, W2, b2):
    src = edge_index[0]
    dst = edge_index[1]
    zeros = jnp.zeros((_NP, _D), jnp.float32)
    agg2 = _sc_agg(x, src, dst, zeros)
    return _tc_mlp(x, agg2, W1, b1, W2, b2)


# speedup vs baseline: 13.8096x; 1.1742x over previous
import functools

import jax
import jax.numpy as jnp
from jax import lax
from jax.experimental import pallas as pl
from jax.experimental.pallas import tpu as pltpu
from jax.experimental.pallas import tpu_sc as plsc

_N = 10000
_NP = 10240
_E = 320000
_D = 128
_NC = 2
_NS = 16
_CH = 80
_NB = 3


def _make_sc_agg():
    mesh = plsc.VectorSubcoreMesh(core_axis_name="c", subcore_axis_name="s")
    n_workers = _NC * _NS
    epw = _E // n_workers
    n_chunks = epw // _CH
    rows_per_tile = _NP // _NS

    rpt0 = 624
    rpt1 = _N - (_NS - 1) * rpt0

    @functools.partial(
        pl.kernel,
        mesh=mesh,
        out_type=jax.ShapeDtypeStruct((_NC, _N, _D), jnp.float32),
        scratch_types=[
            pltpu.VMEM((2 * epw,), jnp.int32),
            pltpu.VMEM((_CH,), jnp.int32),
            pltpu.VMEM((_NB, _CH, _D), jnp.float32),
            pltpu.VMEM_SHARED((_N, _D), jnp.float32),
            pltpu.SemaphoreType.DMA,
            pltpu.SemaphoreType.DMA,
        ] + [pltpu.SemaphoreType.DMA] * _NB,
    )
    def sc_agg(x_hbm, src_hbm, dst_hbm, zeros_hbm, out_hbm,
               idx_v, dstc, rows, agg_sh, zsem, isem, *gsems):
        gsems = list(gsems)
        cid = lax.axis_index("c")
        sid = lax.axis_index("s")
        wid = sid * _NC + cid
        last = sid == _NS - 1
        r0 = sid * rpt0

        def rng(ref):
            return (ref.at[pl.ds(r0, rpt0)],
                    ref.at[pl.ds((_NS - 1) * rpt0, rpt1)])

        zsrc0, zsrc1 = rng(zeros_hbm)
        zdst0, zdst1 = rng(agg_sh)

        @pl.when(~last)
        def _():
            pltpu.async_copy(zsrc0, zdst0, zsem)

        @pl.when(last)
        def _():
            pltpu.async_copy(zsrc1, zdst1, zsem)

        base = wid * epw
        pltpu.async_copy(src_hbm.at[pl.ds(base, epw)],
                         idx_v.at[pl.ds(0, epw)], isem)
        pltpu.async_copy(dst_hbm.at[pl.ds(base, epw)],
                         idx_v.at[pl.ds(epw, epw)], isem)

        @pl.when(~last)
        def _():
            pltpu.make_async_copy(zsrc0, zdst0, zsem).wait()

        @pl.when(last)
        def _():
            pltpu.make_async_copy(zsrc1, zdst1, zsem).wait()

        pltpu.make_async_copy(src_hbm.at[pl.ds(base, epw)],
                              idx_v.at[pl.ds(0, epw)], isem).wait()
        pltpu.make_async_copy(src_hbm.at[pl.ds(base, epw)],
                              idx_v.at[pl.ds(0, epw)], isem).wait()
        plsc.subcore_barrier()

        def sl(c):
            return pl.ds(pl.multiple_of(c * _CH, _CH), _CH)

        def copy_dst(c):
            off = pl.multiple_of(epw + c * _CH, _CH)
            for j in range(_CH // 16):
                dstc[pl.ds(16 * j, 16)] = idx_v[pl.ds(off + 16 * j, 16)]

        def issue(c, k):
            pltpu.async_copy(x_hbm.at[idx_v.at[sl(c)]], rows.at[k], gsems[k])

        def step(c, k, issue_next):
            pltpu.make_async_copy(x_hbm.at[idx_v.at[sl(0)]], rows.at[k],
                                  gsems[k]).wait()
            copy_dst(c)
            pltpu.sync_copy(rows.at[k], agg_sh.at[dstc], add=True)
            if issue_next:
                @pl.when(c + _NB < n_chunks)
                def _():
                    issue(c + _NB, k)

        for k in range(_NB):
            issue(k, k)

        def body(i, carry):
            c0 = i * _NB
            for k in range(_NB):
                step(c0 + k, k, True)
            return carry

        lax.fori_loop(0, n_chunks // _NB, body, 0)
        for t in range(n_chunks - _NB * (n_chunks // _NB)):
            step(n_chunks - n_chunks % _NB + t, t, False)
        plsc.subcore_barrier()

        @pl.when(~last)
        def _():
            pltpu.sync_copy(agg_sh.at[pl.ds(r0, rpt0)],
                            out_hbm.at[cid, pl.ds(r0, rpt0)])

        @pl.when(last)
        def _():
            pltpu.sync_copy(agg_sh.at[pl.ds((_NS - 1) * rpt0, rpt1)],
                            out_hbm.at[cid, pl.ds((_NS - 1) * rpt0, rpt1)])

    return sc_agg


_sc_agg = _make_sc_agg()

_BLK = 1000


def _mlp_body(x_ref, a_ref, w1_ref, b1_ref, w2_ref, b2_ref, o_ref):
    h = x_ref[...] + a_ref[0] + a_ref[1]
    h = jnp.dot(h, w1_ref[...], preferred_element_type=jnp.float32) + b1_ref[...]
    h = jnp.maximum(h, 0.01 * h)
    h = jnp.dot(h, w2_ref[...], preferred_element_type=jnp.float32) + b2_ref[...]
    o_ref[...] = jnp.maximum(h, 0.01 * h)


def _tc_mlp(x, agg2, W1, b1, W2, b2):
    return pl.pallas_call(
        _mlp_body,
        grid=(_N // _BLK,),
        in_specs=[
            pl.BlockSpec((_BLK, _D), lambda i: (i, 0)),
            pl.BlockSpec((_NC, _BLK, _D), lambda i: (0, i, 0)),
            pl.BlockSpec((_D, _D), lambda i: (0, 0)),
            pl.BlockSpec((1, _D), lambda i: (0, 0)),
            pl.BlockSpec((_D, _D), lambda i: (0, 0)),
            pl.BlockSpec((1, _D), lambda i: (0, 0)),
        ],
        out_specs=pl.BlockSpec((_BLK, _D), lambda i: (i, 0)),
        out_shape=jax.ShapeDtypeStruct((_N, _D), jnp.float32),
    )(x, agg2, W1, b1.reshape(1, _D), W2, b2.reshape(1, _D))


def kernel(x, edge_index, W1, b1, W2, b2):
    src = edge_index[0]
    dst = edge_index[1]
    zeros = jnp.zeros((_N, _D), jnp.float32)
    agg2 = _sc_agg(x, src, dst, zeros)
    return _tc_mlp(x, agg2, W1, b1, W2, b2)
